# baseline (speedup 1.0000x reference)
import functools

import jax
import jax.numpy as jnp
from jax import lax
from jax.experimental import pallas as pl
from jax.experimental.pallas import tpu as pltpu
from jax.experimental.pallas import tpu_sc as plsc

S = 2048
D_MODEL = 768
D_FF = 3072
E = 8
TOPK = 2
TM = 256
MPAD = S * TOPK + E * TM
NT = MPAD // TM
NF = 2
FFB = D_FF // NF

NC = 2
NS = 16
NW = NC * NS
LANES = 16
DPACK = D_MODEL // 2

GROWS = MPAD // NW
GCH = 96
GNCH = GROWS // GCH
CROWS = S // NW


def _wid():
    return lax.axis_index("s") * NC + lax.axis_index("c")


def _sc_dispatch_body(src_hbm, xi_hbm, xs_hbm, idx_v, buf0, buf1,
                      g0, g1, w0, w1):
    base = _wid() * GROWS
    bufs = (buf0, buf1)
    gsem = (g0, g1)
    wsem = (w0, w1)

    for c in range(GNCH):
        pltpu.sync_copy(src_hbm.at[pl.ds(base + c * GCH, GCH)], idx_v.at[c])
    gathers = [
        pltpu.make_async_copy(xi_hbm.at[idx_v.at[c]], bufs[c], gsem[c])
        for c in range(GNCH)
    ]
    writebacks = [
        pltpu.make_async_copy(
            bufs[c], xs_hbm.at[pl.ds(base + c * GCH, GCH)], wsem[c])
        for c in range(GNCH)
    ]
    for gth in gathers:
        gth.start()
    for c in range(GNCH):
        gathers[c].wait()
        writebacks[c].start()
    for wb in writebacks:
        wb.wait()


@functools.cache
def _sc_dispatch_kernel():
    return pl.kernel(
        _sc_dispatch_body,
        out_type=jax.ShapeDtypeStruct((MPAD, DPACK), jnp.int32),
        mesh=plsc.VectorSubcoreMesh(core_axis_name="c", subcore_axis_name="s"),
        scratch_types=[
            pltpu.VMEM((GNCH, GCH), jnp.int32),
            pltpu.VMEM((GCH, DPACK), jnp.int32),
            pltpu.VMEM((GCH, DPACK), jnp.int32),
            pltpu.SemaphoreType.DMA,
            pltpu.SemaphoreType.DMA,
            pltpu.SemaphoreType.DMA,
            pltpu.SemaphoreType.DMA,
        ],
    )


def _sc_combine_body(p0_hbm, p1_hbm, ys_hbm, y_hbm,
                     i0_v, i1_v, a_v, b_v, sa, sb, sw):
    base = _wid() * CROWS

    pltpu.sync_copy(p0_hbm.at[pl.ds(base, CROWS)], i0_v)
    pltpu.sync_copy(p1_hbm.at[pl.ds(base, CROWS)], i1_v)
    ga = pltpu.make_async_copy(ys_hbm.at[i0_v], a_v, sa)
    gb = pltpu.make_async_copy(ys_hbm.at[i1_v], b_v, sb)
    ga.start()
    gb.start()
    ga.wait()
    gb.wait()

    def row(r, rc):
        for col in range(D_MODEL // LANES):
            sl = pl.ds(col * LANES, LANES)
            a_v[r, sl] = a_v[r, sl] + b_v[r, sl]
        return rc

    lax.fori_loop(0, CROWS, row, 0)
    wb = pltpu.make_async_copy(a_v, y_hbm.at[pl.ds(base, CROWS)], sw)
    wb.start()
    wb.wait()


@functools.cache
def _sc_combine_kernel():
    return pl.kernel(
        _sc_combine_body,
        out_type=jax.ShapeDtypeStruct((S, D_MODEL), jnp.float32),
        mesh=plsc.VectorSubcoreMesh(core_axis_name="c", subcore_axis_name="s"),
        scratch_types=[
            pltpu.VMEM((CROWS,), jnp.int32),
            pltpu.VMEM((CROWS,), jnp.int32),
            pltpu.VMEM((CROWS, D_MODEL), jnp.float32),
            pltpu.VMEM((CROWS, D_MODEL), jnp.float32),
            pltpu.SemaphoreType.DMA,
            pltpu.SemaphoreType.DMA,
            pltpu.SemaphoreType.DMA,
        ],
    )


def _ffn_body(g_ref, u_ref, xs_ref, w1_ref, b1_ref, w2_ref, b2_ref, ws_ref,
              ys_ref):
    i = pl.program_id(0)
    f = pl.program_id(1)

    @pl.when(i < u_ref[0])
    def _compute():
        xb = xs_ref[...]
        w1 = w1_ref[0].astype(jnp.bfloat16)
        h = lax.dot_general(xb, w1, (((1,), (1,)), ((), ())),
                            preferred_element_type=jnp.float32)
        h = jnp.maximum(h + b1_ref[0, 0][None, :], 0.0).astype(jnp.bfloat16)
        w2 = w2_ref[0].astype(jnp.bfloat16)
        o = lax.dot_general(h, w2, (((1,), (1,)), ((), ())),
                            preferred_element_type=jnp.float32)
        b2 = jnp.where(f == 0, b2_ref[0, 0], 0.0)
        o = (o + b2[None, :]) * ws_ref[0, 0][:, None]

        @pl.when(f == 0)
        def _set():
            ys_ref[...] = o

        @pl.when(f != 0)
        def _acc():
            ys_ref[...] += o


@jax.jit
def _grouped_ffn(g, u, xs, w1, b1, w2, b2, ws):
    grid_spec = pltpu.PrefetchScalarGridSpec(
        num_scalar_prefetch=2,
        grid=(NT, NF),
        in_specs=[
            pl.BlockSpec((TM, D_MODEL), lambda i, f, g, u: (i, 0)),
            pl.BlockSpec((1, FFB, D_MODEL), lambda i, f, g, u: (g[i], f, 0)),
            pl.BlockSpec((1, 1, FFB), lambda i, f, g, u: (g[i], 0, f)),
            pl.BlockSpec((1, D_MODEL, FFB), lambda i, f, g, u: (g[i], 0, f)),
            pl.BlockSpec((1, 1, D_MODEL), lambda i, f, g, u: (g[i], 0, 0)),
            pl.BlockSpec((1, 1, TM), lambda i, f, g, u: (i, 0, 0)),
        ],
        out_specs=pl.BlockSpec((TM, D_MODEL), lambda i, f, g, u: (i, 0)),
    )
    return pl.pallas_call(
        _ffn_body,
        grid_spec=grid_spec,
        out_shape=jax.ShapeDtypeStruct((MPAD, D_MODEL), jnp.float32),
        compiler_params=pltpu.CompilerParams(
            dimension_semantics=("arbitrary", "arbitrary"),
        ),
    )(g, u, xs, w1, b1, w2, b2, ws)


def _routing(topk_probs, topk_idx):
    e0 = topk_idx[:, 0]
    e1 = topk_idx[:, 1]
    memb = (jax.nn.one_hot(e0, E, dtype=jnp.int32)
            + jax.nn.one_hot(e1, E, dtype=jnp.int32))
    cum = jnp.cumsum(memb, axis=0)
    counts = cum[-1]
    excl = cum - memb
    cnt_pad = ((counts + TM - 1) // TM) * TM
    bound = jnp.cumsum(cnt_pad)
    astart = bound - cnt_pad
    tarange = jnp.arange(S, dtype=jnp.int32)
    p0 = astart[e0] + excl[tarange, e0]
    p1 = astart[e1] + excl[tarange, e1]

    mflat = jnp.concatenate([p0, p1])
    tok = jnp.concatenate([tarange, tarange])
    src = jnp.zeros((MPAD,), jnp.int32).at[mflat].set(tok)
    ws = jnp.zeros((MPAD,), jnp.float32).at[mflat].set(
        jnp.concatenate([topk_probs[:, 0], topk_probs[:, 1]]))

    nused = (bound[-1] // TM).astype(jnp.int32)
    tile_start = jnp.arange(NT, dtype=jnp.int32) * TM
    g = jnp.searchsorted(bound, tile_start, side='right').astype(jnp.int32)
    g = jnp.where(jnp.arange(NT) < nused, jnp.minimum(g, E - 1),
                  jnp.minimum(g[jnp.maximum(nused - 1, 0)], E - 1))
    return src, ws, p0, p1, g, nused


def kernel(x, gate_w, w1, b1, w2, b2):
    s, b, h = x.shape
    x_flat = x.reshape(s * b, h)

    logits = x_flat @ gate_w.T
    probs = jax.nn.softmax(logits, axis=-1)
    topk_probs, topk_idx = jax.lax.top_k(probs, TOPK)

    src, ws, p0, p1, g, nused = _routing(topk_probs, topk_idx)

    xi = lax.bitcast_convert_type(
        x_flat.astype(jnp.bfloat16).reshape(s * b, DPACK, 2), jnp.int32)
    xs_i32 = _sc_dispatch_kernel()(src, xi)
    xs = lax.bitcast_convert_type(xs_i32, jnp.bfloat16).reshape(MPAD, D_MODEL)

    ys = _grouped_ffn(
        g, nused.reshape(1), xs, w1,
        b1.reshape(E, 1, D_FF), w2, b2.reshape(E, 1, D_MODEL),
        ws.reshape(NT, 1, TM),
    )
    y_flat = _sc_combine_kernel()(p0, p1, ys)
    return y_flat.reshape(s, b, h)


# speedup vs baseline: 2.0096x; 2.0096x over previous
import functools

import jax
import jax.numpy as jnp
from jax import lax
from jax.experimental import pallas as pl
from jax.experimental.pallas import tpu as pltpu
from jax.experimental.pallas import tpu_sc as plsc

S = 2048
D_MODEL = 768
D_FF = 3072
E = 8
TOPK = 2
TM = 256
MPAD = S * TOPK + E * TM
NT = MPAD // TM

NC = 2
NS = 16
NW = NC * NS
LANES = 16
CROWS = S // NW


def _wid():
    return lax.axis_index("s") * NC + lax.axis_index("c")


def _sc_combine_body(p0_hbm, p1_hbm, ys_hbm, y_hbm,
                     i0_v, i1_v, a_v, b_v, sa, sb, sw):
    base = _wid() * CROWS

    pltpu.sync_copy(p0_hbm.at[pl.ds(base, CROWS)], i0_v)
    pltpu.sync_copy(p1_hbm.at[pl.ds(base, CROWS)], i1_v)
    ga = pltpu.make_async_copy(ys_hbm.at[i0_v], a_v, sa)
    gb = pltpu.make_async_copy(ys_hbm.at[i1_v], b_v, sb)
    ga.start()
    gb.start()
    ga.wait()
    gb.wait()

    def row(r, rc):
        for col in range(D_MODEL // LANES):
            sl = pl.ds(col * LANES, LANES)
            a_v[r, sl] = a_v[r, sl] + b_v[r, sl]
        return rc

    lax.fori_loop(0, CROWS, row, 0)
    wb = pltpu.make_async_copy(a_v, y_hbm.at[pl.ds(base, CROWS)], sw)
    wb.start()
    wb.wait()


@functools.cache
def _sc_combine_kernel():
    return pl.kernel(
        _sc_combine_body,
        out_type=jax.ShapeDtypeStruct((S, D_MODEL), jnp.float32),
        mesh=plsc.VectorSubcoreMesh(core_axis_name="c", subcore_axis_name="s"),
        scratch_types=[
            pltpu.VMEM((CROWS,), jnp.int32),
            pltpu.VMEM((CROWS,), jnp.int32),
            pltpu.VMEM((CROWS, D_MODEL), jnp.float32),
            pltpu.VMEM((CROWS, D_MODEL), jnp.float32),
            pltpu.SemaphoreType.DMA,
            pltpu.SemaphoreType.DMA,
            pltpu.SemaphoreType.DMA,
        ],
    )


def _ffn_body(g_ref, u_ref, src_ref, x_ref, w1_ref, b1_ref, w2_ref, b2_ref,
              ws_ref, ys_ref):
    i = pl.program_id(0)

    @pl.when(i < u_ref[0])
    def _compute():
        sid = src_ref[0, 0]
        onehot = (lax.broadcasted_iota(jnp.int32, (TM, S), 1)
                  == sid[:, None]).astype(jnp.bfloat16)
        xg = lax.dot_general(onehot, x_ref[...], (((1,), (0,)), ((), ())),
                             preferred_element_type=jnp.float32
                             ).astype(jnp.bfloat16)

        w1 = w1_ref[0].astype(jnp.bfloat16)
        h = lax.dot_general(xg, w1, (((1,), (1,)), ((), ())),
                            preferred_element_type=jnp.float32)
        h = jnp.maximum(h + b1_ref[0, 0][None, :], 0.0).astype(jnp.bfloat16)
        w2 = w2_ref[0].astype(jnp.bfloat16)
        o = lax.dot_general(h, w2, (((1,), (1,)), ((), ())),
                            preferred_element_type=jnp.float32)
        o = o + b2_ref[0, 0][None, :]
        ys_ref[...] = o * ws_ref[0, 0][:, None]


@jax.jit
def _grouped_ffn(g, u, src, xb, w1, b1, w2, b2, ws):
    grid_spec = pltpu.PrefetchScalarGridSpec(
        num_scalar_prefetch=2,
        grid=(NT,),
        in_specs=[
            pl.BlockSpec((1, 1, TM), lambda i, g, u: (i, 0, 0)),
            pl.BlockSpec((S, D_MODEL), lambda i, g, u: (0, 0)),
            pl.BlockSpec((1, D_FF, D_MODEL), lambda i, g, u: (g[i], 0, 0)),
            pl.BlockSpec((1, 1, D_FF), lambda i, g, u: (g[i], 0, 0)),
            pl.BlockSpec((1, D_MODEL, D_FF), lambda i, g, u: (g[i], 0, 0)),
            pl.BlockSpec((1, 1, D_MODEL), lambda i, g, u: (g[i], 0, 0)),
            pl.BlockSpec((1, 1, TM), lambda i, g, u: (i, 0, 0)),
        ],
        out_specs=pl.BlockSpec((TM, D_MODEL), lambda i, g, u: (i, 0)),
    )
    return pl.pallas_call(
        _ffn_body,
        grid_spec=grid_spec,
        out_shape=jax.ShapeDtypeStruct((MPAD, D_MODEL), jnp.float32),
        compiler_params=pltpu.CompilerParams(
            dimension_semantics=("arbitrary",),
        ),
    )(g, u, src, xb, w1, b1, w2, b2, ws)


def _routing(topk_probs, topk_idx):
    e0 = topk_idx[:, 0]
    e1 = topk_idx[:, 1]
    memb = (jax.nn.one_hot(e0, E, dtype=jnp.int32)
            + jax.nn.one_hot(e1, E, dtype=jnp.int32))
    cum = jnp.cumsum(memb, axis=0)
    counts = cum[-1]
    excl = cum - memb
    cnt_pad = ((counts + TM - 1) // TM) * TM
    bound = jnp.cumsum(cnt_pad)
    astart = bound - cnt_pad
    tarange = jnp.arange(S, dtype=jnp.int32)
    p0 = astart[e0] + excl[tarange, e0]
    p1 = astart[e1] + excl[tarange, e1]

    mflat = jnp.concatenate([p0, p1])
    tok = jnp.concatenate([tarange, tarange])
    src = jnp.zeros((MPAD,), jnp.int32).at[mflat].set(tok)
    ws = jnp.zeros((MPAD,), jnp.float32).at[mflat].set(
        jnp.concatenate([topk_probs[:, 0], topk_probs[:, 1]]))

    nused = (bound[-1] // TM).astype(jnp.int32)
    tile_start = jnp.arange(NT, dtype=jnp.int32) * TM
    g = jnp.searchsorted(bound, tile_start, side='right').astype(jnp.int32)
    g = jnp.where(jnp.arange(NT) < nused, jnp.minimum(g, E - 1),
                  jnp.minimum(g[jnp.maximum(nused - 1, 0)], E - 1))
    return src, ws, p0, p1, g, nused


def kernel(x, gate_w, w1, b1, w2, b2):
    s, b, h = x.shape
    x_flat = x.reshape(s * b, h)

    logits = x_flat @ gate_w.T
    probs = jax.nn.softmax(logits, axis=-1)
    topk_probs, topk_idx = jax.lax.top_k(probs, TOPK)

    src, ws, p0, p1, g, nused = _routing(topk_probs, topk_idx)

    ys = _grouped_ffn(
        g, nused.reshape(1), src.reshape(NT, 1, TM),
        x_flat.astype(jnp.bfloat16), w1,
        b1.reshape(E, 1, D_FF), w2, b2.reshape(E, 1, D_MODEL),
        ws.reshape(NT, 1, TM),
    )
    y_flat = _sc_combine_kernel()(p0, p1, ys)
    return y_flat.reshape(s, b, h)


# speedup vs baseline: 2.5138x; 1.2509x over previous
import functools

import jax
import jax.numpy as jnp
from jax import lax
from jax.experimental import pallas as pl
from jax.experimental.pallas import tpu as pltpu
from jax.experimental.pallas import tpu_sc as plsc

S = 2048
D_MODEL = 768
D_FF = 3072
E = 8
TOPK = 2
TM = 256
MPAD = S * TOPK + E * TM
NT = MPAD // TM

NC = 2
NS = 16
NW = NC * NS
LANES = 16
CROWS = S // NW


def _wid():
    return lax.axis_index("s") * NC + lax.axis_index("c")


def _sc_combine_body(p0_hbm, p1_hbm, ys_hbm, y_hbm,
                     i0_v, i1_v, a_v, b_v, sa, sb, sw):
    base = _wid() * CROWS

    pltpu.sync_copy(p0_hbm.at[pl.ds(base, CROWS)], i0_v)
    pltpu.sync_copy(p1_hbm.at[pl.ds(base, CROWS)], i1_v)
    ga = pltpu.make_async_copy(ys_hbm.at[i0_v], a_v, sa)
    gb = pltpu.make_async_copy(ys_hbm.at[i1_v], b_v, sb)
    ga.start()
    gb.start()
    ga.wait()
    gb.wait()

    def row(r, rc):
        for col in range(D_MODEL // LANES):
            sl = pl.ds(col * LANES, LANES)
            a_v[r, sl] = a_v[r, sl] + b_v[r, sl]
        return rc

    lax.fori_loop(0, CROWS, row, 0)
    wb = pltpu.make_async_copy(a_v, y_hbm.at[pl.ds(base, CROWS)], sw)
    wb.start()
    wb.wait()


@functools.cache
def _sc_combine_kernel():
    return pl.kernel(
        _sc_combine_body,
        out_type=jax.ShapeDtypeStruct((S, D_MODEL), jnp.float32),
        mesh=plsc.VectorSubcoreMesh(core_axis_name="c", subcore_axis_name="s"),
        scratch_types=[
            pltpu.VMEM((CROWS,), jnp.int32),
            pltpu.VMEM((CROWS,), jnp.int32),
            pltpu.VMEM((CROWS, D_MODEL), jnp.float32),
            pltpu.VMEM((CROWS, D_MODEL), jnp.float32),
            pltpu.SemaphoreType.DMA,
            pltpu.SemaphoreType.DMA,
            pltpu.SemaphoreType.DMA,
        ],
    )


def _ffn_body(g_ref, u_ref, p0_ref, p1_ref, q0_ref, q1_ref, x_ref,
              w1_ref, b1_ref, w2_ref, b2_ref, ys_ref):
    i = pl.program_id(0)

    @pl.when(i < u_ref[0])
    def _compute():
        slot = i * TM + lax.broadcasted_iota(jnp.int32, (TM, 1), 0)
        m0 = p0_ref[...] == slot
        m1 = p1_ref[...] == slot
        onehot = jnp.logical_or(m0, m1).astype(jnp.bfloat16)
        wsl = (jnp.sum(jnp.where(m0, q0_ref[...], 0.0), axis=1)
               + jnp.sum(jnp.where(m1, q1_ref[...], 0.0), axis=1))
        xg = lax.dot_general(onehot, x_ref[...], (((1,), (0,)), ((), ())),
                             preferred_element_type=jnp.float32
                             ).astype(jnp.bfloat16)

        w1 = w1_ref[0].astype(jnp.bfloat16)
        h = lax.dot_general(xg, w1, (((1,), (1,)), ((), ())),
                            preferred_element_type=jnp.float32)
        h = jnp.maximum(h + b1_ref[0, 0][None, :], 0.0).astype(jnp.bfloat16)
        w2 = w2_ref[0].astype(jnp.bfloat16)
        o = lax.dot_general(h, w2, (((1,), (1,)), ((), ())),
                            preferred_element_type=jnp.float32)
        o = o + b2_ref[0, 0][None, :]
        ys_ref[...] = o * wsl[:, None]


@jax.jit
def _grouped_ffn(g, u, p0, p1, q0, q1, xb, w1, b1, w2, b2):
    grid_spec = pltpu.PrefetchScalarGridSpec(
        num_scalar_prefetch=2,
        grid=(NT,),
        in_specs=[
            pl.BlockSpec((1, S), lambda i, g, u: (0, 0)),
            pl.BlockSpec((1, S), lambda i, g, u: (0, 0)),
            pl.BlockSpec((1, S), lambda i, g, u: (0, 0)),
            pl.BlockSpec((1, S), lambda i, g, u: (0, 0)),
            pl.BlockSpec((S, D_MODEL), lambda i, g, u: (0, 0)),
            pl.BlockSpec((1, D_FF, D_MODEL), lambda i, g, u: (g[i], 0, 0)),
            pl.BlockSpec((1, 1, D_FF), lambda i, g, u: (g[i], 0, 0)),
            pl.BlockSpec((1, D_MODEL, D_FF), lambda i, g, u: (g[i], 0, 0)),
            pl.BlockSpec((1, 1, D_MODEL), lambda i, g, u: (g[i], 0, 0)),
        ],
        out_specs=pl.BlockSpec((TM, D_MODEL), lambda i, g, u: (i, 0)),
    )
    return pl.pallas_call(
        _ffn_body,
        grid_spec=grid_spec,
        out_shape=jax.ShapeDtypeStruct((MPAD, D_MODEL), jnp.float32),
        compiler_params=pltpu.CompilerParams(
            dimension_semantics=("arbitrary",),
        ),
    )(g, u, p0, p1, q0, q1, xb, w1, b1, w2, b2)


def _routing(topk_probs, topk_idx):
    e0 = topk_idx[:, 0]
    e1 = topk_idx[:, 1]
    oh0 = jax.nn.one_hot(e0, E, dtype=jnp.int32)
    oh1 = jax.nn.one_hot(e1, E, dtype=jnp.int32)
    memb = oh0 + oh1
    cum = jnp.cumsum(memb, axis=0)
    counts = cum[-1]
    excl = cum - memb
    cnt_pad = ((counts + TM - 1) // TM) * TM
    bound = jnp.cumsum(cnt_pad)
    astart = bound - cnt_pad
    slot0 = astart[None, :] + excl
    p0 = jnp.sum(slot0 * oh0, axis=1)
    p1 = jnp.sum(slot0 * oh1, axis=1)

    nused = (bound[-1] // TM).astype(jnp.int32)
    tile_start = jnp.arange(NT, dtype=jnp.int32) * TM
    g = jnp.sum((tile_start[:, None] >= bound[None, :]).astype(jnp.int32),
                axis=1)
    g = jnp.where(jnp.arange(NT) < nused, jnp.minimum(g, E - 1),
                  jnp.minimum(g[jnp.maximum(nused - 1, 0)], E - 1))
    return p0, p1, g, nused


def kernel(x, gate_w, w1, b1, w2, b2):
    s, b, h = x.shape
    x_flat = x.reshape(s * b, h)

    logits = x_flat @ gate_w.T
    probs = jax.nn.softmax(logits, axis=-1)
    topk_probs, topk_idx = jax.lax.top_k(probs, TOPK)

    p0, p1, g, nused = _routing(topk_probs, topk_idx)

    ys = _grouped_ffn(
        g, nused.reshape(1), p0.reshape(1, S), p1.reshape(1, S),
        topk_probs[:, 0].reshape(1, S), topk_probs[:, 1].reshape(1, S),
        x_flat.astype(jnp.bfloat16), w1,
        b1.reshape(E, 1, D_FF), w2, b2.reshape(E, 1, D_MODEL),
    )
    y_flat = _sc_combine_kernel()(p0, p1, ys)
    return y_flat.reshape(s, b, h)


# speedup vs baseline: 2.5429x; 1.0116x over previous
import functools

import jax
import jax.numpy as jnp
from jax import lax
from jax.experimental import pallas as pl
from jax.experimental.pallas import tpu as pltpu
from jax.experimental.pallas import tpu_sc as plsc

S = 2048
D_MODEL = 768
D_FF = 3072
E = 8
TOPK = 2
TM = 256
MPAD = S * TOPK + E * TM
NT = MPAD // TM

NC = 2
NS = 16
NW = NC * NS
LANES = 16
CROWS = S // NW


def _wid():
    return lax.axis_index("s") * NC + lax.axis_index("c")


def _sc_combine_body(p0_hbm, p1_hbm, ys_hbm, y_hbm,
                     i0_v, i1_v, a_v, b_v, sa, sb, sw):
    base = _wid() * CROWS

    pltpu.sync_copy(p0_hbm.at[pl.ds(base, CROWS)], i0_v)
    pltpu.sync_copy(p1_hbm.at[pl.ds(base, CROWS)], i1_v)
    ga = pltpu.make_async_copy(ys_hbm.at[i0_v], a_v, sa)
    gb = pltpu.make_async_copy(ys_hbm.at[i1_v], b_v, sb)
    ga.start()
    gb.start()
    ga.wait()
    gb.wait()

    def row(r, rc):
        for col in range(D_MODEL // LANES):
            sl = pl.ds(col * LANES, LANES)
            a_v[r, sl] = a_v[r, sl] + b_v[r, sl]
        return rc

    lax.fori_loop(0, CROWS, row, 0)
    wb = pltpu.make_async_copy(a_v, y_hbm.at[pl.ds(base, CROWS)], sw)
    wb.start()
    wb.wait()


@functools.cache
def _sc_combine_kernel():
    return pl.kernel(
        _sc_combine_body,
        out_type=jax.ShapeDtypeStruct((S, D_MODEL), jnp.float32),
        mesh=plsc.VectorSubcoreMesh(core_axis_name="c", subcore_axis_name="s"),
        scratch_types=[
            pltpu.VMEM((CROWS,), jnp.int32),
            pltpu.VMEM((CROWS,), jnp.int32),
            pltpu.VMEM((CROWS, D_MODEL), jnp.float32),
            pltpu.VMEM((CROWS, D_MODEL), jnp.float32),
            pltpu.SemaphoreType.DMA,
            pltpu.SemaphoreType.DMA,
            pltpu.SemaphoreType.DMA,
        ],
    )


def _ffn_body(g_ref, u_ref, p0_ref, p1_ref, q0_ref, q1_ref, x_ref,
              w1_ref, b1_ref, w2_ref, b2_ref, ys_ref):
    i = pl.program_id(0)

    @pl.when(i < u_ref[0])
    def _compute():
        slot = i * TM + lax.broadcasted_iota(jnp.int32, (TM, 1), 0)
        m0 = p0_ref[...] == slot
        m1 = p1_ref[...] == slot
        onehot = jnp.logical_or(m0, m1).astype(jnp.bfloat16)
        wsl = (jnp.sum(jnp.where(m0, q0_ref[...], 0.0), axis=1)
               + jnp.sum(jnp.where(m1, q1_ref[...], 0.0), axis=1))
        xg = lax.dot_general(onehot, x_ref[...], (((1,), (0,)), ((), ())),
                             preferred_element_type=jnp.float32
                             ).astype(jnp.bfloat16)

        w1 = w1_ref[0].astype(jnp.bfloat16)
        h = lax.dot_general(xg, w1, (((1,), (1,)), ((), ())),
                            preferred_element_type=jnp.float32)
        h = jnp.maximum(h + b1_ref[0, 0][None, :], 0.0).astype(jnp.bfloat16)
        w2 = w2_ref[0].astype(jnp.bfloat16)
        o = lax.dot_general(h, w2, (((1,), (1,)), ((), ())),
                            preferred_element_type=jnp.float32)
        o = o + b2_ref[0, 0][None, :]
        ys_ref[...] = o * wsl[:, None]


@jax.jit
def _grouped_ffn(g, u, p0, p1, q0, q1, xb, w1, b1, w2, b2):
    grid_spec = pltpu.PrefetchScalarGridSpec(
        num_scalar_prefetch=2,
        grid=(NT,),
        in_specs=[
            pl.BlockSpec((1, S), lambda i, g, u: (0, 0)),
            pl.BlockSpec((1, S), lambda i, g, u: (0, 0)),
            pl.BlockSpec((1, S), lambda i, g, u: (0, 0)),
            pl.BlockSpec((1, S), lambda i, g, u: (0, 0)),
            pl.BlockSpec((S, D_MODEL), lambda i, g, u: (0, 0)),
            pl.BlockSpec((1, D_FF, D_MODEL), lambda i, g, u: (g[i], 0, 0)),
            pl.BlockSpec((1, 1, D_FF), lambda i, g, u: (g[i], 0, 0)),
            pl.BlockSpec((1, D_MODEL, D_FF), lambda i, g, u: (g[i], 0, 0)),
            pl.BlockSpec((1, 1, D_MODEL), lambda i, g, u: (g[i], 0, 0)),
        ],
        out_specs=pl.BlockSpec((TM, D_MODEL), lambda i, g, u: (i, 0)),
    )
    return pl.pallas_call(
        _ffn_body,
        grid_spec=grid_spec,
        out_shape=jax.ShapeDtypeStruct((MPAD, D_MODEL), jnp.float32),
        compiler_params=pltpu.CompilerParams(
            dimension_semantics=("arbitrary",),
        ),
    )(g, u, p0, p1, q0, q1, xb, w1, b1, w2, b2)


def _routing(oh0, oh1):
    memb = oh0 + oh1
    cum = jnp.cumsum(memb, axis=0)
    counts = cum[-1]
    excl = cum - memb
    cnt_pad = ((counts + TM - 1) // TM) * TM
    bound = jnp.cumsum(cnt_pad)
    astart = bound - cnt_pad
    slot0 = astart[None, :] + excl
    p0 = jnp.sum(slot0 * oh0, axis=1)
    p1 = jnp.sum(slot0 * oh1, axis=1)

    nused = (bound[-1] // TM).astype(jnp.int32)
    tile_start = jnp.arange(NT, dtype=jnp.int32) * TM
    g = jnp.sum((tile_start[:, None] >= bound[None, :]).astype(jnp.int32),
                axis=1)
    g = jnp.where(jnp.arange(NT) < nused, jnp.minimum(g, E - 1),
                  jnp.minimum(g[jnp.maximum(nused - 1, 0)], E - 1))
    return p0, p1, g, nused


def kernel(x, gate_w, w1, b1, w2, b2):
    s, b, h = x.shape
    x_flat = x.reshape(s * b, h)

    logits = x_flat @ gate_w.T
    probs = jax.nn.softmax(logits, axis=-1)
    q0 = jnp.max(probs, axis=1)
    earange = jnp.arange(E, dtype=jnp.int32)[None, :]
    oh0 = probs == q0[:, None]
    oh0 = jnp.logical_and(oh0, jnp.cumsum(oh0, axis=1) == 1)
    probs_m = jnp.where(oh0, -1.0, probs)
    q1 = jnp.max(probs_m, axis=1)
    oh1 = probs_m == q1[:, None]
    oh1 = jnp.logical_and(oh1, jnp.cumsum(oh1, axis=1) == 1)

    p0, p1, g, nused = _routing(oh0.astype(jnp.int32), oh1.astype(jnp.int32))

    ys = _grouped_ffn(
        g, nused.reshape(1), p0.reshape(1, S), p1.reshape(1, S),
        q0.reshape(1, S), q1.reshape(1, S),
        x_flat.astype(jnp.bfloat16), w1,
        b1.reshape(E, 1, D_FF), w2, b2.reshape(E, 1, D_MODEL),
    )
    y_flat = _sc_combine_kernel()(p0, p1, ys)
    return y_flat.reshape(s, b, h)


# speedup vs baseline: 2.6931x; 1.0590x over previous
import functools

import jax
import jax.numpy as jnp
from jax import lax
from jax.experimental import pallas as pl
from jax.experimental.pallas import tpu as pltpu

S = 2048
D_MODEL = 768
D_FF = 3072
E = 8
TOPK = 2
TM = 256
MPAD = S * TOPK + E * TM
NT = MPAD // TM


def _ffn_body(g_ref, u_ref, p0r_ref, p1r_ref, p0c_ref, p1c_ref, q0_ref,
              q1_ref, x_ref, w1_ref, b1_ref, w2_ref, b2_ref, y_ref):
    i = pl.program_id(0)

    @pl.when(i == 0)
    def _init():
        y_ref[...] = jnp.zeros_like(y_ref)

    @pl.when(i < u_ref[0])
    def _compute():
        slot = i * TM + lax.broadcasted_iota(jnp.int32, (TM, 1), 0)
        m0 = p0r_ref[...] == slot
        m1 = p1r_ref[...] == slot
        onehot = jnp.logical_or(m0, m1).astype(jnp.bfloat16)
        wsl = (jnp.sum(jnp.where(m0, q0_ref[...], 0.0), axis=1)
               + jnp.sum(jnp.where(m1, q1_ref[...], 0.0), axis=1))
        xg = lax.dot_general(onehot, x_ref[...], (((1,), (0,)), ((), ())),
                             preferred_element_type=jnp.float32
                             ).astype(jnp.bfloat16)

        w1 = w1_ref[0].astype(jnp.bfloat16)
        h = lax.dot_general(xg, w1, (((1,), (1,)), ((), ())),
                            preferred_element_type=jnp.float32)
        h = jnp.maximum(h + b1_ref[0, 0][None, :], 0.0).astype(jnp.bfloat16)
        w2 = w2_ref[0].astype(jnp.bfloat16)
        o = lax.dot_general(h, w2, (((1,), (1,)), ((), ())),
                            preferred_element_type=jnp.float32)
        o = ((o + b2_ref[0, 0][None, :]) * wsl[:, None]).astype(jnp.bfloat16)

        slot_row = i * TM + lax.broadcasted_iota(jnp.int32, (1, TM), 1)
        back = jnp.logical_or(p0c_ref[...] == slot_row,
                              p1c_ref[...] == slot_row).astype(jnp.bfloat16)
        y_ref[...] += lax.dot_general(back, o, (((1,), (0,)), ((), ())),
                                      preferred_element_type=jnp.float32)


@jax.jit
def _grouped_ffn(g, u, p0r, p1r, p0c, p1c, q0, q1, xb, w1, b1, w2, b2):
    grid_spec = pltpu.PrefetchScalarGridSpec(
        num_scalar_prefetch=2,
        grid=(NT,),
        in_specs=[
            pl.BlockSpec((1, S), lambda i, g, u: (0, 0)),
            pl.BlockSpec((1, S), lambda i, g, u: (0, 0)),
            pl.BlockSpec((S, 1), lambda i, g, u: (0, 0)),
            pl.BlockSpec((S, 1), lambda i, g, u: (0, 0)),
            pl.BlockSpec((1, S), lambda i, g, u: (0, 0)),
            pl.BlockSpec((1, S), lambda i, g, u: (0, 0)),
            pl.BlockSpec((S, D_MODEL), lambda i, g, u: (0, 0)),
            pl.BlockSpec((1, D_FF, D_MODEL), lambda i, g, u: (g[i], 0, 0)),
            pl.BlockSpec((1, 1, D_FF), lambda i, g, u: (g[i], 0, 0)),
            pl.BlockSpec((1, D_MODEL, D_FF), lambda i, g, u: (g[i], 0, 0)),
            pl.BlockSpec((1, 1, D_MODEL), lambda i, g, u: (g[i], 0, 0)),
        ],
        out_specs=pl.BlockSpec((S, D_MODEL), lambda i, g, u: (0, 0)),
    )
    return pl.pallas_call(
        _ffn_body,
        grid_spec=grid_spec,
        out_shape=jax.ShapeDtypeStruct((S, D_MODEL), jnp.float32),
        compiler_params=pltpu.CompilerParams(
            dimension_semantics=("arbitrary",),
        ),
    )(g, u, p0r, p1r, p0c, p1c, q0, q1, xb, w1, b1, w2, b2)


def _routing(oh0, oh1):
    memb = oh0 + oh1
    cum = jnp.cumsum(memb, axis=0)
    counts = cum[-1]
    excl = cum - memb
    cnt_pad = ((counts + TM - 1) // TM) * TM
    bound = jnp.cumsum(cnt_pad)
    astart = bound - cnt_pad
    slot0 = astart[None, :] + excl
    p0 = jnp.sum(slot0 * oh0, axis=1)
    p1 = jnp.sum(slot0 * oh1, axis=1)

    nused = (bound[-1] // TM).astype(jnp.int32)
    tile_start = jnp.arange(NT, dtype=jnp.int32) * TM
    g = jnp.sum((tile_start[:, None] >= bound[None, :]).astype(jnp.int32),
                axis=1)
    g = jnp.where(jnp.arange(NT) < nused, jnp.minimum(g, E - 1),
                  jnp.minimum(g[jnp.maximum(nused - 1, 0)], E - 1))
    return p0, p1, g, nused


def kernel(x, gate_w, w1, b1, w2, b2):
    s, b, h = x.shape
    x_flat = x.reshape(s * b, h)

    logits = x_flat @ gate_w.T
    probs = jax.nn.softmax(logits, axis=-1)
    q0 = jnp.max(probs, axis=1)
    oh0 = probs == q0[:, None]
    oh0 = jnp.logical_and(oh0, jnp.cumsum(oh0, axis=1) == 1)
    probs_m = jnp.where(oh0, -1.0, probs)
    q1 = jnp.max(probs_m, axis=1)
    oh1 = probs_m == q1[:, None]
    oh1 = jnp.logical_and(oh1, jnp.cumsum(oh1, axis=1) == 1)

    p0, p1, g, nused = _routing(oh0.astype(jnp.int32), oh1.astype(jnp.int32))

    y_flat = _grouped_ffn(
        g, nused.reshape(1), p0.reshape(1, S), p1.reshape(1, S),
        p0.reshape(S, 1), p1.reshape(S, 1),
        q0.reshape(1, S), q1.reshape(1, S),
        x_flat.astype(jnp.bfloat16), w1,
        b1.reshape(E, 1, D_FF), w2, b2.reshape(E, 1, D_MODEL),
    )
    return y_flat.reshape(s, b, h)
